# baseline (speedup 1.0000x reference)
import jax
import jax.numpy as jnp
from jax.experimental import pallas as pl
from jax.experimental.pallas import tpu as pltpu

B, L = 8192, 8192
BLOCK_ROWS = 64


def _kernel(a_ref, b_ref, o_ref):
    s = jnp.sum(a_ref[...], axis=1, keepdims=True)
    o_ref[...] = b_ref[...] * s


def kernel(a, b):
    grid = (B // BLOCK_ROWS,)
    return pl.pallas_call(
        _kernel,
        grid=grid,
        in_specs=[
            pl.BlockSpec((BLOCK_ROWS, L), lambda i: (i, 0)),
            pl.BlockSpec((BLOCK_ROWS, L), lambda i: (i, 0)),
        ],
        out_specs=pl.BlockSpec((BLOCK_ROWS, L), lambda i: (i, 0)),
        out_shape=jax.ShapeDtypeStruct((B, L), jnp.float32),
        compiler_params=pltpu.CompilerParams(
            dimension_semantics=("parallel",),
        ),
    )(a, b)


# speedup vs baseline: 1.0292x; 1.0292x over previous
import jax
import jax.numpy as jnp
from jax.experimental import pallas as pl
from jax.experimental.pallas import tpu as pltpu

B, L = 8192, 8192
BLOCK_ROWS = 128


def _kernel(a_ref, b_ref, o_ref):
    s = jnp.sum(a_ref[...], axis=1, keepdims=True)
    o_ref[...] = b_ref[...] * s


def kernel(a, b):
    grid = (B // BLOCK_ROWS,)
    return pl.pallas_call(
        _kernel,
        grid=grid,
        in_specs=[
            pl.BlockSpec((BLOCK_ROWS, L), lambda i: (i, 0)),
            pl.BlockSpec((BLOCK_ROWS, L), lambda i: (i, 0)),
        ],
        out_specs=pl.BlockSpec((BLOCK_ROWS, L), lambda i: (i, 0)),
        out_shape=jax.ShapeDtypeStruct((B, L), jnp.float32),
        compiler_params=pltpu.CompilerParams(
            dimension_semantics=("parallel",),
        ),
    )(a, b)
